# baseline (speedup 1.0000x reference)
import functools

import jax
import jax.numpy as jnp
from jax import lax
from jax.experimental import pallas as pl
from jax.experimental.pallas import tpu as pltpu
from jax.experimental.pallas import tpu_sc as plsc

K_NEIGH = 5
ALPHA = 1.0

B = 1024
F = 512
NBANK = 100000
C = 64
CHUNK = 2048
NCHUNK = 49
NPAD = NCHUNK * CHUNK
GW = 128
GPC = CHUNK // GW
NG = NPAD // GW
NGP = 896
NSEL = K_NEIGH + 1
JPAD = (NBANK - (NCHUNK - 1) * CHUNK) // GW

_NEG = -3.0e38
_BIGI = 2**30

_SC_NC = 2
_SC_NS = 16
_SC_NW = _SC_NC * _SC_NS


def _topk_iter(vals, idx, k):
    maxs, sels = [], []
    for _ in range(k):
        m = jnp.max(vals, axis=1, keepdims=True)
        cand = jnp.where(vals == m, idx, _BIGI)
        sel = jnp.min(cand, axis=1, keepdims=True)
        maxs.append(m)
        sels.append(sel)
        vals = jnp.where(idx == sel, _NEG, vals)
    return maxs, sels


def _k1_body(feat_ref, bank_ref, dist_ref, gsel_ref, fidx_ref,
             fn_ref, gm_ref, pend_ref):
    i = pl.program_id(0)

    @pl.when(i == 0)
    def _():
        f = feat_ref[...]
        nrm = jnp.sqrt(jnp.sum(f * f, axis=1, keepdims=True))
        fn_ref[...] = f / jnp.maximum(nrm, 1e-12)

    scores = lax.dot_general(
        fn_ref[...], bank_ref[...], (((1,), (1,)), ((), ())),
        preferred_element_type=jnp.float32,
    )

    parts = []
    for j in range(GPC):
        blk = scores[:, j * GW:(j + 1) * GW]
        dist_ref[j] = blk
        parts.append(jnp.max(blk, axis=1, keepdims=True))
    nv = jnp.concatenate(parts, axis=1)

    for k in range(8):
        @pl.when(lax.rem(i, 8) == k)
        def _(k=k):
            pend_ref[:, k * GPC:(k + 1) * GPC] = nv

    @pl.when((lax.rem(i, 8) == 7) & (i != NCHUNK - 1))
    def _():
        gm_ref[:, pl.ds((i // 8) * 128, 128)] = pend_ref[...]

    @pl.when(i == NCHUNK - 1)
    def _():
        for j in range(JPAD, GPC):
            col = (jax.lax.broadcasted_iota(jnp.int32, (B, GW), 1)
                   + (NCHUNK - 1) * CHUNK + j * GW)
            blkm = jnp.where(col < NBANK, scores[:, j * GW:(j + 1) * GW],
                             _NEG)
            dist_ref[j] = blkm
            pend_ref[:, j:j + 1] = jnp.max(blkm, axis=1, keepdims=True)
        gm_ref[:, (NGP - 128):NGP] = pend_ref[...]

        gidx = jax.lax.broadcasted_iota(jnp.int32, (B, NGP), 1)
        vals = jnp.where(gidx < NG, gm_ref[...], _NEG)
        _, sels = _topk_iter(vals, gidx, NSEL)
        g6 = jnp.concatenate(sels + [sels[-1], sels[-1]], axis=1)
        gsel_ref[...] = g6
        row = jax.lax.broadcasted_iota(jnp.int32, (B, 8), 0)
        fidx_ref[...] = g6 * B + row


def _k1_call(features, fea_bank):
    return pl.pallas_call(
        _k1_body,
        grid=(NCHUNK,),
        in_specs=[
            pl.BlockSpec((B, F), lambda i: (0, 0)),
            pl.BlockSpec((CHUNK, F), lambda i: (i, 0)),
        ],
        out_specs=[
            pl.BlockSpec((GPC, B, GW), lambda i: (i, 0, 0)),
            pl.BlockSpec((B, 8), lambda i: (0, 0)),
            pl.BlockSpec((B, 8), lambda i: (0, 0)),
        ],
        out_shape=[
            jax.ShapeDtypeStruct((NG, B, GW), jnp.float32),
            jax.ShapeDtypeStruct((B, 8), jnp.int32),
            jax.ShapeDtypeStruct((B, 8), jnp.int32),
        ],
        scratch_shapes=[
            pltpu.VMEM((B, F), jnp.float32),
            pltpu.VMEM((B, NGP), jnp.float32),
            pltpu.VMEM((B, 128), jnp.float32),
        ],
    )(features, fea_bank)


def _sc_gather(table, idx, D):
    n = idx.shape[0]
    bpw = n // _SC_NW
    mesh = plsc.VectorSubcoreMesh(core_axis_name="c", subcore_axis_name="s")

    @functools.partial(
        pl.kernel,
        out_type=jax.ShapeDtypeStruct((n, D), jnp.float32),
        mesh=mesh,
        scratch_types=[
            pltpu.VMEM((bpw,), jnp.int32),
            pltpu.VMEM((bpw, D), jnp.float32),
            pltpu.SemaphoreType.DMA,
        ],
    )
    def k(table_hbm, idx_hbm, out_hbm, idx_v, rows_v, sem):
        wid = lax.axis_index("s") * _SC_NC + lax.axis_index("c")
        base = wid * bpw
        pltpu.sync_copy(idx_hbm.at[pl.ds(base, bpw)], idx_v)
        pltpu.async_copy(table_hbm.at[idx_v], rows_v, sem).wait()
        pltpu.sync_copy(rows_v, out_hbm.at[pl.ds(base, bpw)])

    return k(table, idx)


def _k4_body(cand_ref, gsel_ref, out_ref):
    cand = cand_ref[...]
    lane = jax.lax.broadcasted_iota(jnp.int32, (B, GW), 1)
    eidx = jnp.concatenate(
        [gsel_ref[:, j:j + 1] * GW + lane for j in range(NSEL)], axis=1
    )
    _, sels = _topk_iter(cand, eidx, NSEL)
    out_ref[...] = jnp.concatenate(sels + [sels[-1], sels[-1]], axis=1)


def _k4_call(cand, gsel):
    return pl.pallas_call(
        _k4_body,
        out_shape=jax.ShapeDtypeStruct((B, 8), jnp.int32),
    )(cand, gsel)


def _k6_body(sn2_ref, idx8_ref, logits_ref, out_ref):
    lg = logits_ref[...]
    m = jnp.max(lg, axis=1, keepdims=True)
    e = jnp.exp(lg - m)
    q = e / jnp.sum(e, axis=1, keepdims=True)
    sn2 = sn2_ref[...]
    par = lax.rem(idx8_ref[:, 1:1 + K_NEIGH], 2)
    sn = jnp.where(par[:, :, None] == 0, sn2[:, :, :C], sn2[:, :, C:])
    kl = sn * (jnp.log(sn) - q[:, None, :])
    kl_sum = jnp.sum(jnp.sum(jnp.sum(kl, axis=2), axis=1))
    colsum = jnp.sum(q, axis=0, keepdims=True)
    t1 = jnp.sum(colsum * colsum)
    t2 = jnp.sum(q * q)
    out_ref[0, 0] = kl_sum / B + (t1 - t2) / B * ALPHA


def _k6_call(sn2, idx8, logits):
    return pl.pallas_call(
        _k6_body,
        out_shape=jax.ShapeDtypeStruct((1, 1), jnp.float32),
        out_specs=pl.BlockSpec(memory_space=pltpu.SMEM),
    )(sn2, idx8, logits)


def kernel(features, fea_bank, score_bank, logits):
    dist, gsel, fidx = _k1_call(features, fea_bank)
    groups = dist.reshape(NG * B, GW)
    cand = _sc_gather(groups, fidx[:, :NSEL].reshape(-1), GW)
    idx8 = _k4_call(cand.reshape(B, NSEL * GW), gsel)
    idx_near = idx8[:, 1:1 + K_NEIGH].reshape(-1)
    st = score_bank.reshape(NBANK // 2, 2 * C)
    sn2 = _sc_gather(st, idx_near // 2, 2 * C)
    loss = _k6_call(sn2.reshape(B, K_NEIGH, 2 * C), idx8, logits)
    return loss[0, 0]


# speedup vs baseline: 1.0463x; 1.0463x over previous
import functools

import jax
import jax.numpy as jnp
from jax import lax
from jax.experimental import pallas as pl
from jax.experimental.pallas import tpu as pltpu
from jax.experimental.pallas import tpu_sc as plsc

K_NEIGH = 5
ALPHA = 1.0

B = 1024
F = 512
NBANK = 100000
C = 64
CHUNK = 2048
NCHUNK = 49
NPAD = NCHUNK * CHUNK
GW = 128
GPC = CHUNK // GW
NG = NPAD // GW
NGP = 896
NSEL = K_NEIGH + 1
JPAD = (NBANK - (NCHUNK - 1) * CHUNK) // GW

_NEG = -3.0e38
_BIGI = 2**30

_SC_NC = 2
_SC_NS = 16
_SC_NW = _SC_NC * _SC_NS
_QPW = B // _SC_NW


def _topk_iter(vals, idx, k):
    maxs, sels = [], []
    for _ in range(k):
        m = jnp.max(vals, axis=1, keepdims=True)
        cand = jnp.where(vals == m, idx, _BIGI)
        sel = jnp.min(cand, axis=1, keepdims=True)
        maxs.append(m)
        sels.append(sel)
        vals = jnp.where(idx == sel, _NEG, vals)
    return maxs, sels


def _k1_body(feat_ref, bank_ref, dist_ref, gsel_ref, fidx_ref,
             fn_ref, gm_ref, pend_ref):
    i = pl.program_id(0)

    @pl.when(i == 0)
    def _():
        f = feat_ref[...]
        nrm = jnp.sqrt(jnp.sum(f * f, axis=1, keepdims=True))
        fn_ref[...] = f / jnp.maximum(nrm, 1e-12)

    scores = lax.dot_general(
        fn_ref[...], bank_ref[...], (((1,), (1,)), ((), ())),
        preferred_element_type=jnp.float32,
    )

    parts = []
    for j in range(GPC):
        blk = scores[:, j * GW:(j + 1) * GW]
        dist_ref[j] = blk
        parts.append(jnp.max(blk, axis=1, keepdims=True))
    nv = jnp.concatenate(parts, axis=1)

    for k in range(8):
        @pl.when(lax.rem(i, 8) == k)
        def _(k=k):
            pend_ref[:, k * GPC:(k + 1) * GPC] = nv

    @pl.when((lax.rem(i, 8) == 7) & (i != NCHUNK - 1))
    def _():
        gm_ref[:, pl.ds((i // 8) * 128, 128)] = pend_ref[...]

    @pl.when(i == NCHUNK - 1)
    def _():
        for j in range(JPAD, GPC):
            col = (jax.lax.broadcasted_iota(jnp.int32, (B, GW), 1)
                   + (NCHUNK - 1) * CHUNK + j * GW)
            blkm = jnp.where(col < NBANK, scores[:, j * GW:(j + 1) * GW],
                             _NEG)
            dist_ref[j] = blkm
            pend_ref[:, j:j + 1] = jnp.max(blkm, axis=1, keepdims=True)
        gm_ref[:, (NGP - 128):NGP] = pend_ref[...]

        gidx = jax.lax.broadcasted_iota(jnp.int32, (B, NGP), 1)
        vals = jnp.where(gidx < NG, gm_ref[...], _NEG)
        _, sels = _topk_iter(vals, gidx, NSEL)
        g6 = jnp.concatenate(sels + [sels[-1], sels[-1]], axis=1)
        gsel_ref[...] = g6
        row = jax.lax.broadcasted_iota(jnp.int32, (B, 8), 0)
        fidx_ref[...] = (g6 * B + row).T


def _k1_call(features, fea_bank):
    return pl.pallas_call(
        _k1_body,
        grid=(NCHUNK,),
        in_specs=[
            pl.BlockSpec((B, F), lambda i: (0, 0)),
            pl.BlockSpec((CHUNK, F), lambda i: (i, 0)),
        ],
        out_specs=[
            pl.BlockSpec((GPC, B, GW), lambda i: (i, 0, 0)),
            pl.BlockSpec((B, 8), lambda i: (0, 0)),
            pl.BlockSpec((8, B), lambda i: (0, 0)),
        ],
        out_shape=[
            jax.ShapeDtypeStruct((NG, B, GW), jnp.float32),
            jax.ShapeDtypeStruct((B, 8), jnp.int32),
            jax.ShapeDtypeStruct((8, B), jnp.int32),
        ],
        scratch_shapes=[
            pltpu.VMEM((B, F), jnp.float32),
            pltpu.VMEM((B, NGP), jnp.float32),
            pltpu.VMEM((B, 128), jnp.float32),
        ],
    )(features, fea_bank)


def _sc_gather_ranks(table, idxT, first, nsel, D):
    mesh = plsc.VectorSubcoreMesh(core_axis_name="c", subcore_axis_name="s")

    @functools.partial(
        pl.kernel,
        out_type=jax.ShapeDtypeStruct((nsel, B, D), jnp.float32),
        mesh=mesh,
        scratch_types=[
            pltpu.VMEM((nsel, _QPW), jnp.int32),
            pltpu.VMEM((nsel, _QPW, D), jnp.float32),
            pltpu.SemaphoreType.DMA,
        ],
    )
    def k(table_hbm, idx_hbm, out_hbm, idx_v, rows_v, sem):
        wid = lax.axis_index("s") * _SC_NC + lax.axis_index("c")
        qb = wid * _QPW
        for j in range(nsel):
            pltpu.sync_copy(idx_hbm.at[first + j, pl.ds(qb, _QPW)],
                            idx_v.at[j])
        cps = [
            pltpu.async_copy(table_hbm.at[idx_v.at[j]], rows_v.at[j], sem)
            for j in range(nsel)
        ]
        for cp in cps:
            cp.wait()
        pltpu.sync_copy(rows_v,
                        out_hbm.at[pl.ds(0, nsel), pl.ds(qb, _QPW)])

    return k(table, idxT)


def _k4_body(cand_ref, gsel_ref, idx8_ref, idxhT_ref):
    lane = jax.lax.broadcasted_iota(jnp.int32, (B, GW), 1)
    cand = jnp.concatenate([cand_ref[j] for j in range(NSEL)], axis=1)
    eidx = jnp.concatenate(
        [gsel_ref[:, j:j + 1] * GW + lane for j in range(NSEL)], axis=1
    )
    _, sels = _topk_iter(cand, eidx, NSEL)
    idx8 = jnp.concatenate(sels + [sels[-1], sels[-1]], axis=1)
    idx8_ref[...] = idx8
    idxhT_ref[...] = (idx8 // 2).T


def _k4_call(cand, gsel):
    return pl.pallas_call(
        _k4_body,
        out_shape=[
            jax.ShapeDtypeStruct((B, 8), jnp.int32),
            jax.ShapeDtypeStruct((8, B), jnp.int32),
        ],
    )(cand, gsel)


def _k6_body(sn2_ref, idx8_ref, logits_ref, out_ref):
    lg = logits_ref[...]
    m = jnp.max(lg, axis=1, keepdims=True)
    e = jnp.exp(lg - m)
    q = e / jnp.sum(e, axis=1, keepdims=True)
    par = lax.rem(idx8_ref[:, 1:1 + K_NEIGH], 2)
    kl_sum = jnp.float32(0.0)
    for k in range(K_NEIGH):
        sk = sn2_ref[k]
        snk = jnp.where(par[:, k:k + 1] == 0, sk[:, :C], sk[:, C:])
        kl_sum += jnp.sum(snk * (jnp.log(snk) - q))
    colsum = jnp.sum(q, axis=0, keepdims=True)
    t1 = jnp.sum(colsum * colsum)
    t2 = jnp.sum(q * q)
    out_ref[0, 0] = kl_sum / B + (t1 - t2) / B * ALPHA


def _k6_call(sn2, idx8, logits):
    return pl.pallas_call(
        _k6_body,
        out_shape=jax.ShapeDtypeStruct((1, 1), jnp.float32),
        out_specs=pl.BlockSpec(memory_space=pltpu.SMEM),
    )(sn2, idx8, logits)


def kernel(features, fea_bank, score_bank, logits):
    dist, gsel, fidxT = _k1_call(features, fea_bank)
    groups = dist.reshape(NG * B, GW)
    cand = _sc_gather_ranks(groups, fidxT, 0, NSEL, GW)
    idx8, idxhT = _k4_call(cand, gsel)
    st = score_bank.reshape(NBANK // 2, 2 * C)
    sn2 = _sc_gather_ranks(st, idxhT, 1, K_NEIGH, 2 * C)
    loss = _k6_call(sn2, idx8, logits)
    return loss[0, 0]
